# Initial kernel scaffold; baseline (speedup 1.0000x reference)
import jax
import jax.numpy as jnp
from jax.experimental import pallas as pl


def kernel(X, node_idx, hedge_idx, W, b):
    raise NotImplementedError("write your pallas kernel here")


# speedup vs baseline: 1.0974x; 1.0974x over previous
import functools

import jax
import jax.numpy as jnp
from jax import lax
from jax.experimental import pallas as pl
from jax.experimental.pallas import tpu as pltpu, tpu_sc as plsc

_N_NODES = 10000
_N_HEDGES = 5000
_N_NP = 10240
_N_HP = 5120
_D = 128
_NC, _NS = 2, 16
_NW = _NC * _NS
_CH = 32
_EPT = 10240

_mesh = plsc.VectorSubcoreMesh(core_axis_name="c", subcore_axis_name="s")


def _gather_body(table_hbm, gi_hbm, out_hbm,
                 gi_v0, gi_v1, rows_v0, rows_v1, sem0, sem1):
    cid = lax.axis_index("c")
    sid = lax.axis_index("s")
    wid = cid * _NS + sid
    base = wid * _EPT

    def body(j, carry):
        b = base + j * (2 * _CH)
        pltpu.sync_copy(gi_hbm.at[pl.ds(b, _CH)], gi_v0)
        h0 = pltpu.async_copy(table_hbm.at[gi_v0], rows_v0, sem0)
        pltpu.sync_copy(gi_hbm.at[pl.ds(b + _CH, _CH)], gi_v1)
        h1 = pltpu.async_copy(table_hbm.at[gi_v1], rows_v1, sem1)
        h0.wait()
        pltpu.sync_copy(rows_v0, out_hbm.at[pl.ds(b, _CH)])
        h1.wait()
        pltpu.sync_copy(rows_v1, out_hbm.at[pl.ds(b + _CH, _CH)])
        return carry

    lax.fori_loop(0, _EPT // (2 * _CH), body, 0)


_gather = pl.kernel(
    _gather_body,
    out_type=jax.ShapeDtypeStruct((_NW * _EPT, _D), jnp.float32),
    mesh=_mesh,
    scratch_types=[
        pltpu.VMEM((_CH,), jnp.int32),
        pltpu.VMEM((_CH,), jnp.int32),
        pltpu.VMEM((_CH, _D), jnp.float32),
        pltpu.VMEM((_CH, _D), jnp.float32),
        pltpu.SemaphoreType.DMA,
        pltpu.SemaphoreType.DMA,
    ],
)


def _proj_body(x_ref, w_ref, b_ref, dv_ref, out_ref):
    xt = jnp.dot(x_ref[...], w_ref[...],
                 preferred_element_type=jnp.float32) + b_ref[...]
    dv = dv_ref[:, 0:1]
    scale = jnp.where(dv > 0, lax.rsqrt(jnp.maximum(dv, 1e-12)), 0.0)
    out_ref[...] = xt * scale


def _escale_body(ef_ref, de_ref, out_ref):
    de = de_ref[:, 0:1]
    inv = jnp.where(de > 0, 1.0 / jnp.maximum(de, 1e-12), 0.0)
    out_ref[...] = ef_ref[...] * inv


def _final_body(o_ref, dv_ref, out_ref):
    dv = dv_ref[:, 0:1]
    scale = jnp.where(dv > 0, lax.rsqrt(jnp.maximum(dv, 1e-12)), 0.0)
    out_ref[...] = jnp.maximum(o_ref[...] * scale, 0.0)


def kernel(X, node_idx, hedge_idx, W, b):
    nnz = node_idx.shape[0]
    npad = _NW * _EPT - nnz
    f32 = jnp.float32
    i32 = jnp.int32
    nidx = jnp.concatenate([node_idx, jnp.full((npad,), _N_NP - 1, i32)])
    hidx = jnp.concatenate([hedge_idx, jnp.full((npad,), _N_HP - 1, i32)])

    ones = jnp.ones((nidx.shape[0],), f32)
    Dv = jax.ops.segment_sum(ones, nidx, num_segments=_N_NP)
    De = jax.ops.segment_sum(ones, hidx, num_segments=_N_HP)
    dv16 = jnp.broadcast_to(Dv[:, None], (_N_NP, 16))
    de16 = jnp.broadcast_to(De[:, None], (_N_HP, 16))

    blk = 640
    Xp = jnp.concatenate([X, jnp.zeros((_N_NP - _N_NODES, _D), f32)])
    Xs = pl.pallas_call(
        _proj_body,
        grid=(_N_NP // blk,),
        in_specs=[
            pl.BlockSpec((blk, _D), lambda i: (i, 0)),
            pl.BlockSpec((_D, _D), lambda i: (0, 0)),
            pl.BlockSpec((1, _D), lambda i: (0, 0)),
            pl.BlockSpec((blk, 16), lambda i: (i, 0)),
        ],
        out_specs=pl.BlockSpec((blk, _D), lambda i: (i, 0)),
        out_shape=jax.ShapeDtypeStruct((_N_NP, _D), f32),
    )(Xp, W, b.reshape(1, _D), dv16)

    gathered = _gather(Xs, nidx)
    ef_raw = jax.ops.segment_sum(gathered, hidx, num_segments=_N_HP)

    ef = pl.pallas_call(
        _escale_body,
        grid=(_N_HP // blk,),
        in_specs=[
            pl.BlockSpec((blk, _D), lambda i: (i, 0)),
            pl.BlockSpec((blk, 16), lambda i: (i, 0)),
        ],
        out_specs=pl.BlockSpec((blk, _D), lambda i: (i, 0)),
        out_shape=jax.ShapeDtypeStruct((_N_HP, _D), f32),
    )(ef_raw, de16)

    back = _gather(ef, hidx)
    out_raw = jax.ops.segment_sum(back, nidx, num_segments=_N_NP)

    out = pl.pallas_call(
        _final_body,
        grid=(_N_NODES // 1000,),
        in_specs=[
            pl.BlockSpec((1000, _D), lambda i: (i, 0)),
            pl.BlockSpec((1000, 16), lambda i: (i, 0)),
        ],
        out_specs=pl.BlockSpec((1000, _D), lambda i: (i, 0)),
        out_shape=jax.ShapeDtypeStruct((_N_NODES, _D), f32),
    )(out_raw, dv16)
    return out
